# baseline (speedup 1.0000x reference)
import jax
import jax.numpy as jnp
from jax.experimental import pallas as pl
from jax.experimental.pallas import tpu as pltpu

_HB = 8


def _fused_heads_body(x_ref, wc_ref, bc_ref, wr_ref, br_ref, wd_ref, bd_ref,
                      cls_ref, reg_ref, dir_ref):
    c, hb, w = x_ref.shape[1], x_ref.shape[2], x_ref.shape[3]
    xb = x_ref[0].astype(jnp.bfloat16).reshape(c, hb * w)
    cls_ref[0] = (jnp.dot(wc_ref[...], xb, preferred_element_type=jnp.float32)
                  + bc_ref[...]).reshape(cls_ref.shape[1], hb, w)
    reg_ref[0] = (jnp.dot(wr_ref[...], xb, preferred_element_type=jnp.float32)
                  + br_ref[...]).reshape(reg_ref.shape[1], hb, w)
    dir_ref[0] = (jnp.dot(wd_ref[...], xb, preferred_element_type=jnp.float32)
                  + bd_ref[...]).reshape(dir_ref.shape[1], hb, w)


@jax.jit
def kernel(x, W_cls, b_cls, W_reg, b_reg, W_dir, b_dir):
    B, C, H, W = x.shape

    def _wspec(o):
        return pl.BlockSpec((o, C), lambda b, j: (0, 0))

    def _bspec(o):
        return pl.BlockSpec((o, 1), lambda b, j: (0, 0))

    def _ospec(o):
        return pl.BlockSpec((1, o, _HB, W), lambda b, j: (b, 0, j, 0))

    o_cls, o_reg, o_dir = W_cls.shape[0], W_reg.shape[0], W_dir.shape[0]

    cls_o, reg_o, dir_o = pl.pallas_call(
        _fused_heads_body,
        grid=(B, H // _HB),
        in_specs=[
            pl.BlockSpec((1, C, _HB, W), lambda b, j: (b, 0, j, 0)),
            _wspec(o_cls), _bspec(o_cls),
            _wspec(o_reg), _bspec(o_reg),
            _wspec(o_dir), _bspec(o_dir),
        ],
        out_specs=(_ospec(o_cls), _ospec(o_reg), _ospec(o_dir)),
        out_shape=(
            jax.ShapeDtypeStruct((B, o_cls, H, W), jnp.float32),
            jax.ShapeDtypeStruct((B, o_reg, H, W), jnp.float32),
            jax.ShapeDtypeStruct((B, o_dir, H, W), jnp.float32),
        ),
        compiler_params=pltpu.CompilerParams(
            dimension_semantics=("parallel", "arbitrary")),
    )(x,
      W_cls.astype(jnp.bfloat16), b_cls.reshape(o_cls, 1),
      W_reg.astype(jnp.bfloat16), b_reg.reshape(o_reg, 1),
      W_dir.astype(jnp.bfloat16), b_dir.reshape(o_dir, 1))

    return (cls_o, reg_o, dir_o)


# speedup vs baseline: 2.0301x; 2.0301x over previous
import jax
import jax.numpy as jnp
from jax.experimental import pallas as pl
from jax.experimental.pallas import tpu as pltpu

_HB = 8


def _fused_heads_body(x_ref, wc_ref, bc_ref, wr_ref, br_ref, wd_ref, bd_ref,
                      cls_ref, reg_ref, dir_ref):
    hb, w, c = x_ref.shape[1], x_ref.shape[2], x_ref.shape[3]
    xb = x_ref[0].reshape(hb * w, c)

    def head(w_ref, b_ref, out_ref):
        o = out_ref.shape[1]
        r = jnp.dot(xb, w_ref[...], preferred_element_type=jnp.float32)
        out_ref[0] = (jnp.transpose(r) + b_ref[...]).reshape(o, hb, w)

    head(wc_ref, bc_ref, cls_ref)
    head(wr_ref, br_ref, reg_ref)
    head(wd_ref, bd_ref, dir_ref)


@jax.jit
def kernel(x, W_cls, b_cls, W_reg, b_reg, W_dir, b_dir):
    B, C, H, W = x.shape
    xt = jnp.transpose(x, (0, 2, 3, 1))

    def _wspec(o):
        return pl.BlockSpec((C, o), lambda b, j: (0, 0))

    def _bspec(o):
        return pl.BlockSpec((o, 1), lambda b, j: (0, 0))

    def _ospec(o):
        return pl.BlockSpec((1, o, _HB, W), lambda b, j: (b, 0, j, 0))

    o_cls, o_reg, o_dir = W_cls.shape[0], W_reg.shape[0], W_dir.shape[0]

    cls_o, reg_o, dir_o = pl.pallas_call(
        _fused_heads_body,
        grid=(B, H // _HB),
        in_specs=[
            pl.BlockSpec((1, _HB, W, C), lambda b, j: (b, j, 0, 0)),
            _wspec(o_cls), _bspec(o_cls),
            _wspec(o_reg), _bspec(o_reg),
            _wspec(o_dir), _bspec(o_dir),
        ],
        out_specs=(_ospec(o_cls), _ospec(o_reg), _ospec(o_dir)),
        out_shape=(
            jax.ShapeDtypeStruct((B, o_cls, H, W), jnp.float32),
            jax.ShapeDtypeStruct((B, o_reg, H, W), jnp.float32),
            jax.ShapeDtypeStruct((B, o_dir, H, W), jnp.float32),
        ),
        compiler_params=pltpu.CompilerParams(
            dimension_semantics=("parallel", "arbitrary")),
    )(xt,
      W_cls.T, b_cls.reshape(o_cls, 1),
      W_reg.T, b_reg.reshape(o_reg, 1),
      W_dir.T, b_dir.reshape(o_dir, 1))

    return (cls_o, reg_o, dir_o)


# speedup vs baseline: 2.8249x; 1.3916x over previous
import jax
import jax.numpy as jnp
from jax.experimental import pallas as pl
from jax.experimental.pallas import tpu as pltpu

_HB = 8


def _fused_heads_body(x_ref, w_ref, b_ref, cls_ref, reg_ref, dir_ref):
    hb, w, c = x_ref.shape[1], x_ref.shape[2], x_ref.shape[3]
    xb = x_ref[0].reshape(hb * w, c)
    r = jnp.dot(xb, w_ref[...], preferred_element_type=jnp.float32)
    f = (jnp.transpose(r) + b_ref[...]).reshape(r.shape[1], hb, w)
    o_cls, o_reg = cls_ref.shape[1], reg_ref.shape[1]
    cls_ref[0] = f[:o_cls]
    reg_ref[0] = f[o_cls:o_cls + o_reg]
    dir_ref[0] = f[o_cls + o_reg:]


@jax.jit
def kernel(x, W_cls, b_cls, W_reg, b_reg, W_dir, b_dir):
    B, C, H, W = x.shape
    xt = jnp.transpose(x, (0, 2, 3, 1))

    o_cls, o_reg, o_dir = W_cls.shape[0], W_reg.shape[0], W_dir.shape[0]
    o_all = o_cls + o_reg + o_dir
    w_all = jnp.concatenate([W_cls, W_reg, W_dir], axis=0).T
    b_all = jnp.concatenate([b_cls, b_reg, b_dir]).reshape(o_all, 1)

    def _ospec(o):
        return pl.BlockSpec((1, o, _HB, W), lambda b, j: (b, 0, j, 0))

    cls_o, reg_o, dir_o = pl.pallas_call(
        _fused_heads_body,
        grid=(B, H // _HB),
        in_specs=[
            pl.BlockSpec((1, _HB, W, C), lambda b, j: (b, j, 0, 0)),
            pl.BlockSpec((C, o_all), lambda b, j: (0, 0)),
            pl.BlockSpec((o_all, 1), lambda b, j: (0, 0)),
        ],
        out_specs=(_ospec(o_cls), _ospec(o_reg), _ospec(o_dir)),
        out_shape=(
            jax.ShapeDtypeStruct((B, o_cls, H, W), jnp.float32),
            jax.ShapeDtypeStruct((B, o_reg, H, W), jnp.float32),
            jax.ShapeDtypeStruct((B, o_dir, H, W), jnp.float32),
        ),
        compiler_params=pltpu.CompilerParams(
            dimension_semantics=("parallel", "arbitrary")),
    )(xt, w_all, b_all)

    return (cls_o, reg_o, dir_o)


# speedup vs baseline: 3.7828x; 1.3391x over previous
import jax
import jax.numpy as jnp
from jax.experimental import pallas as pl
from jax.experimental.pallas import tpu as pltpu

_HB = 32


def _fused_heads_body(x_ref, w_ref, b_ref, cls_ref, reg_ref, dir_ref):
    hb, w, c = x_ref.shape[1], x_ref.shape[2], x_ref.shape[3]
    xb = x_ref[0].reshape(hb * w, c)
    r = jnp.dot(xb, w_ref[...], preferred_element_type=jnp.float32)
    f = (jnp.transpose(r) + b_ref[...]).reshape(r.shape[1], hb, w)
    o_cls, o_reg = cls_ref.shape[1], reg_ref.shape[1]
    cls_ref[0] = f[:o_cls]
    reg_ref[0] = f[o_cls:o_cls + o_reg]
    dir_ref[0] = f[o_cls + o_reg:]


@jax.jit
def kernel(x, W_cls, b_cls, W_reg, b_reg, W_dir, b_dir):
    B, C, H, W = x.shape
    xt = jnp.transpose(x, (0, 2, 3, 1))

    o_cls, o_reg, o_dir = W_cls.shape[0], W_reg.shape[0], W_dir.shape[0]
    o_all = o_cls + o_reg + o_dir
    w_all = jnp.concatenate([W_cls, W_reg, W_dir], axis=0).T
    b_all = jnp.concatenate([b_cls, b_reg, b_dir]).reshape(o_all, 1)

    def _ospec(o):
        return pl.BlockSpec((1, o, _HB, W), lambda b, j: (b, 0, j, 0))

    cls_o, reg_o, dir_o = pl.pallas_call(
        _fused_heads_body,
        grid=(B, pl.cdiv(H, _HB)),
        in_specs=[
            pl.BlockSpec((1, _HB, W, C), lambda b, j: (b, j, 0, 0)),
            pl.BlockSpec((C, o_all), lambda b, j: (0, 0)),
            pl.BlockSpec((o_all, 1), lambda b, j: (0, 0)),
        ],
        out_specs=(_ospec(o_cls), _ospec(o_reg), _ospec(o_dir)),
        out_shape=(
            jax.ShapeDtypeStruct((B, o_cls, H, W), jnp.float32),
            jax.ShapeDtypeStruct((B, o_reg, H, W), jnp.float32),
            jax.ShapeDtypeStruct((B, o_dir, H, W), jnp.float32),
        ),
        compiler_params=pltpu.CompilerParams(
            dimension_semantics=("parallel", "arbitrary")),
    )(xt, w_all, b_all)

    return (cls_o, reg_o, dir_o)


# speedup vs baseline: 3.8865x; 1.0274x over previous
import jax
import jax.numpy as jnp
from jax.experimental import pallas as pl
from jax.experimental.pallas import tpu as pltpu

_HB = 64


def _fused_heads_body(x_ref, w_ref, b_ref, cls_ref, reg_ref, dir_ref):
    hb, w, c = x_ref.shape[1], x_ref.shape[2], x_ref.shape[3]
    xb = x_ref[0].reshape(hb * w, c)
    r = jnp.dot(xb, w_ref[...], preferred_element_type=jnp.float32)
    f = (jnp.transpose(r) + b_ref[...]).reshape(r.shape[1], hb, w)
    o_cls, o_reg = cls_ref.shape[1], reg_ref.shape[1]
    cls_ref[0] = f[:o_cls]
    reg_ref[0] = f[o_cls:o_cls + o_reg]
    dir_ref[0] = f[o_cls + o_reg:]


@jax.jit
def kernel(x, W_cls, b_cls, W_reg, b_reg, W_dir, b_dir):
    B, C, H, W = x.shape
    xt = jnp.transpose(x, (0, 2, 3, 1))

    o_cls, o_reg, o_dir = W_cls.shape[0], W_reg.shape[0], W_dir.shape[0]
    o_all = o_cls + o_reg + o_dir
    w_all = jnp.concatenate([W_cls, W_reg, W_dir], axis=0).T
    b_all = jnp.concatenate([b_cls, b_reg, b_dir]).reshape(o_all, 1)

    def _ospec(o):
        return pl.BlockSpec((1, o, _HB, W), lambda b, j: (b, 0, j, 0))

    cls_o, reg_o, dir_o = pl.pallas_call(
        _fused_heads_body,
        grid=(B, pl.cdiv(H, _HB)),
        in_specs=[
            pl.BlockSpec((1, _HB, W, C), lambda b, j: (b, j, 0, 0)),
            pl.BlockSpec((C, o_all), lambda b, j: (0, 0)),
            pl.BlockSpec((o_all, 1), lambda b, j: (0, 0)),
        ],
        out_specs=(_ospec(o_cls), _ospec(o_reg), _ospec(o_dir)),
        out_shape=(
            jax.ShapeDtypeStruct((B, o_cls, H, W), jnp.float32),
            jax.ShapeDtypeStruct((B, o_reg, H, W), jnp.float32),
            jax.ShapeDtypeStruct((B, o_dir, H, W), jnp.float32),
        ),
        compiler_params=pltpu.CompilerParams(
            dimension_semantics=("parallel", "arbitrary")),
    )(xt, w_all, b_all)

    return (cls_o, reg_o, dir_o)


# speedup vs baseline: 3.8913x; 1.0012x over previous
import jax
import jax.numpy as jnp
from jax.experimental import pallas as pl
from jax.experimental.pallas import tpu as pltpu

_WB = 24


def _fused_heads_body(x_ref, w_ref, b_ref, cls_ref, reg_ref, dir_ref):
    h, wb, c = x_ref.shape[1], x_ref.shape[2], x_ref.shape[3]
    xb = x_ref[0].reshape(h * wb, c)
    r = jnp.dot(xb, w_ref[...], preferred_element_type=jnp.float32)
    f = (jnp.transpose(r) + b_ref[...]).reshape(r.shape[1], h, wb)
    ft = jnp.transpose(f, (0, 2, 1))
    o_cls, o_reg = cls_ref.shape[1], reg_ref.shape[1]
    cls_ref[0] = ft[:o_cls]
    reg_ref[0] = ft[o_cls:o_cls + o_reg]
    dir_ref[0] = ft[o_cls + o_reg:]


@jax.jit
def kernel(x, W_cls, b_cls, W_reg, b_reg, W_dir, b_dir):
    B, C, H, W = x.shape
    xt = jnp.transpose(x, (0, 2, 3, 1))

    o_cls, o_reg, o_dir = W_cls.shape[0], W_reg.shape[0], W_dir.shape[0]
    o_all = o_cls + o_reg + o_dir
    w_all = jnp.concatenate([W_cls, W_reg, W_dir], axis=0).T
    b_all = jnp.concatenate([b_cls, b_reg, b_dir]).reshape(o_all, 1)

    def _ospec(o):
        return pl.BlockSpec((1, o, _WB, H), lambda b, j: (b, 0, j, 0))

    cls_o, reg_o, dir_o = pl.pallas_call(
        _fused_heads_body,
        grid=(B, W // _WB),
        in_specs=[
            pl.BlockSpec((1, H, _WB, C), lambda b, j: (b, 0, j, 0)),
            pl.BlockSpec((C, o_all), lambda b, j: (0, 0)),
            pl.BlockSpec((o_all, 1), lambda b, j: (0, 0)),
        ],
        out_specs=(_ospec(o_cls), _ospec(o_reg), _ospec(o_dir)),
        out_shape=(
            jax.ShapeDtypeStruct((B, o_cls, W, H), jnp.float32),
            jax.ShapeDtypeStruct((B, o_reg, W, H), jnp.float32),
            jax.ShapeDtypeStruct((B, o_dir, W, H), jnp.float32),
        ),
        compiler_params=pltpu.CompilerParams(
            dimension_semantics=("parallel", "arbitrary")),
    )(xt, w_all, b_all)

    return (jnp.transpose(cls_o, (0, 1, 3, 2)),
            jnp.transpose(reg_o, (0, 1, 3, 2)),
            jnp.transpose(dir_o, (0, 1, 3, 2)))


# speedup vs baseline: 4.4293x; 1.1383x over previous
import jax
import jax.numpy as jnp
from jax.experimental import pallas as pl
from jax.experimental.pallas import tpu as pltpu

_HB = 128
_WB = 72


def _fused_heads_body(x_ref, w_ref, b_ref, cls_ref, reg_ref, dir_ref):
    hb, wb, c = x_ref.shape[1], x_ref.shape[2], x_ref.shape[3]
    xb = x_ref[0].reshape(hb * wb, c)
    r = jnp.dot(xb, w_ref[...], preferred_element_type=jnp.float32)
    f = (jnp.transpose(r) + b_ref[...]).reshape(r.shape[1], hb, wb)
    ft = jnp.transpose(f, (0, 2, 1))
    o_cls, o_reg = cls_ref.shape[1], reg_ref.shape[1]
    cls_ref[0] = ft[:o_cls]
    reg_ref[0] = ft[o_cls:o_cls + o_reg]
    dir_ref[0] = ft[o_cls + o_reg:]


@jax.jit
def kernel(x, W_cls, b_cls, W_reg, b_reg, W_dir, b_dir):
    B, C, H, W = x.shape
    xt = jnp.transpose(x, (0, 2, 3, 1))

    o_cls, o_reg, o_dir = W_cls.shape[0], W_reg.shape[0], W_dir.shape[0]
    o_all = o_cls + o_reg + o_dir
    w_all = jnp.concatenate([W_cls, W_reg, W_dir], axis=0).T
    b_all = jnp.concatenate([b_cls, b_reg, b_dir]).reshape(o_all, 1)

    def _ospec(o):
        return pl.BlockSpec((1, o, _WB, _HB), lambda b, jh, jw: (b, 0, jw, jh))

    cls_o, reg_o, dir_o = pl.pallas_call(
        _fused_heads_body,
        grid=(B, pl.cdiv(H, _HB), W // _WB),
        in_specs=[
            pl.BlockSpec((1, _HB, _WB, C), lambda b, jh, jw: (b, jh, jw, 0)),
            pl.BlockSpec((C, o_all), lambda b, jh, jw: (0, 0)),
            pl.BlockSpec((o_all, 1), lambda b, jh, jw: (0, 0)),
        ],
        out_specs=(_ospec(o_cls), _ospec(o_reg), _ospec(o_dir)),
        out_shape=(
            jax.ShapeDtypeStruct((B, o_cls, W, H), jnp.float32),
            jax.ShapeDtypeStruct((B, o_reg, W, H), jnp.float32),
            jax.ShapeDtypeStruct((B, o_dir, W, H), jnp.float32),
        ),
        compiler_params=pltpu.CompilerParams(
            dimension_semantics=("parallel", "arbitrary", "arbitrary")),
    )(xt, w_all, b_all)

    return (jnp.transpose(cls_o, (0, 1, 3, 2)),
            jnp.transpose(reg_o, (0, 1, 3, 2)),
            jnp.transpose(dir_o, (0, 1, 3, 2)))
